# Initial kernel scaffold; baseline (speedup 1.0000x reference)
import jax
import jax.numpy as jnp
from jax.experimental import pallas as pl


def kernel(prediction):
    raise NotImplementedError("write your pallas kernel here")


# speedup vs baseline: 11.9202x; 11.9202x over previous
import jax
import jax.numpy as jnp
from jax import lax
from jax.experimental import pallas as pl
from jax.experimental.pallas import tpu as pltpu

_CONF = 0.001
_IOU = 0.6
_MAX_DET = 300
_N = 20000
_P = 20480
_R = _P // 128


def _nms_body(pred_ref, out_ref):
    cx = pred_ref[0]
    cy = pred_ref[1]
    w = pred_ref[2]
    h = pred_ref[3]
    obj = pred_ref[4]
    clsp = pred_ref[5]

    conf = obj * clsp
    cand = (obj > _CONF) & (conf > _CONF)
    s0 = jnp.where(cand, conf, 0.0)
    x1 = cx - w * 0.5
    y1 = cy - h * 0.5
    x2 = cx + w * 0.5
    y2 = cy + h * 0.5
    area = (x2 - x1) * (y2 - y1)

    n = jnp.sum(cand.astype(jnp.int32))

    row_i = lax.broadcasted_iota(jnp.int32, (_R, 128), 0)
    col_i = lax.broadcasted_iota(jnp.int32, (_R, 128), 1)
    flat = row_i * 128 + col_i
    lane = lax.broadcasted_iota(jnp.int32, (1, 128), 1)

    def iou_all(bx1, by1, bx2, by2):
        iw = jnp.maximum(jnp.minimum(x2, bx2) - jnp.maximum(x1, bx1), 0.0)
        ih = jnp.maximum(jnp.minimum(y2, by2) - jnp.maximum(y1, by1), 0.0)
        inter = iw * ih
        barea = (bx2 - bx1) * (by2 - by1)
        return inter / (barea + area - inter)

    def pick_body(i, sw):
        m = jnp.max(sw)
        bi = jnp.min(jnp.where(sw == m, flat, jnp.int32(1 << 30)))
        onehot = flat == bi
        bx1 = jnp.sum(jnp.where(onehot, x1, 0.0))
        by1 = jnp.sum(jnp.where(onehot, y1, 0.0))
        bx2 = jnp.sum(jnp.where(onehot, x2, 0.0))
        by2 = jnp.sum(jnp.where(onehot, y2, 0.0))
        valid = m > _CONF
        iou = iou_all(bx1, by1, bx2, by2)
        sw = jnp.where(iou > _IOU, 0.0, sw)
        row = jnp.where(lane == 0, bx1,
              jnp.where(lane == 1, by1,
              jnp.where(lane == 2, bx2,
              jnp.where(lane == 3, by2,
              jnp.where(lane == 4, m,
              jnp.where(lane == 5, valid.astype(jnp.float32), 0.0))))))
        out_ref[pl.ds(i, 1), :] = row
        return sw

    lax.fori_loop(0, _MAX_DET, pick_body, s0)

    do_merge = (n > 1) & (n < 3000)

    def merge_body(p, _):
        row = out_ref[pl.ds(p, 1), :]
        bx1 = jnp.sum(jnp.where(lane == 0, row, 0.0))
        by1 = jnp.sum(jnp.where(lane == 1, row, 0.0))
        bx2 = jnp.sum(jnp.where(lane == 2, row, 0.0))
        by2 = jnp.sum(jnp.where(lane == 3, row, 0.0))
        sc = jnp.sum(jnp.where(lane == 4, row, 0.0))
        vld = jnp.sum(jnp.where(lane == 5, row, 0.0)) > 0.5
        iou = iou_all(bx1, by1, bx2, by2)
        hit = iou > _IOU
        wgt = jnp.where(hit, s0, 0.0)
        den = jnp.sum(wgt)
        nx1 = jnp.sum(wgt * x1)
        ny1 = jnp.sum(wgt * y1)
        nx2 = jnp.sum(wgt * x2)
        ny2 = jnp.sum(wgt * y2)
        cnt = jnp.sum(jnp.where(hit & (s0 > 0.0), 1.0, 0.0))
        den_s = jnp.where(den > 0.0, den, 1.0)
        fx1 = jnp.where(do_merge, nx1 / den_s, bx1)
        fy1 = jnp.where(do_merge, ny1 / den_s, by1)
        fx2 = jnp.where(do_merge, nx2 / den_s, bx2)
        fy2 = jnp.where(do_merge, ny2 / den_s, by2)
        keep = vld & jnp.where(do_merge, cnt > 1.5, True)
        kf = keep.astype(jnp.float32)
        rowout = jnp.where(lane == 0, fx1,
                 jnp.where(lane == 1, fy1,
                 jnp.where(lane == 2, fx2,
                 jnp.where(lane == 3, fy2,
                 jnp.where(lane == 4, sc, 0.0))))) * kf
        out_ref[pl.ds(p, 1), :] = rowout
        return 0

    lax.fori_loop(0, _MAX_DET, merge_body, 0)


def kernel(prediction):
    x = prediction[0]
    xp = jnp.concatenate(
        [x, jnp.zeros((_P - _N, 6), jnp.float32)], axis=0)
    cols = xp.T.reshape(6, _R, 128)
    out = pl.pallas_call(
        _nms_body,
        out_shape=jax.ShapeDtypeStruct((304, 128), jnp.float32),
    )(cols)
    return out[:_MAX_DET, :6][None]
